# speedup vs baseline: 6.3052x; 6.3052x over previous
import functools

import jax
import jax.numpy as jnp
from jax import lax
from jax.experimental import pallas as pl
from jax.experimental.pallas import tpu as pltpu
from jax.experimental.pallas import tpu_sc as plsc

_NC = 2
_NS = 16
_NW = _NC * _NS
_D = 128
_CHUNK = 200


def _make_sc_gather(n_rows: int):
    rows_per_w = n_rows // _NW
    n_chunks = rows_per_w // _CHUNK
    mesh = plsc.VectorSubcoreMesh(core_axis_name="c", subcore_axis_name="s")

    @functools.partial(
        pl.kernel,
        out_type=jax.ShapeDtypeStruct((n_rows, _D), jnp.float32),
        mesh=mesh,
        scratch_types=[
            pltpu.VMEM((rows_per_w,), jnp.int32),
            pltpu.VMEM((_CHUNK, _D), jnp.float32),
            pltpu.SemaphoreType.DMA,
        ],
    )
    def gather_kernel(ids_hbm, table_hbm, out_hbm, idx_v, rows_v, sem):
        wid = lax.axis_index("s") * _NC + lax.axis_index("c")
        base = wid * rows_per_w
        pltpu.sync_copy(ids_hbm.at[pl.ds(base, rows_per_w)], idx_v)

        def body(g, carry):
            off = g * _CHUNK
            pltpu.async_copy(
                table_hbm.at[idx_v.at[pl.ds(off, _CHUNK)]], rows_v, sem
            ).wait()
            pltpu.sync_copy(rows_v, out_hbm.at[pl.ds(base + off, _CHUNK)])
            return carry

        lax.fori_loop(0, n_chunks, body, 0)

    return gather_kernel


_ROWS_BLK = 1600
_EPS = 1e-12


def _ln_body(x_ref, pos_ref, tt_ref, gamma_ref, beta_ref, o_ref):
    x = x_ref[...].reshape(_ROWS_BLK // _CHUNK, _CHUNK, _D)
    bias = pos_ref[...] + tt_ref[0][None, :]
    h = x + bias[None]
    mean = jnp.mean(h, axis=-1, keepdims=True)
    c = h - mean
    var = jnp.mean(c * c, axis=-1, keepdims=True)
    normed = c * lax.rsqrt(var + _EPS)
    out = normed * gamma_ref[0][None, None, :] + beta_ref[0][None, None, :]
    o_ref[...] = out.reshape(_ROWS_BLK, _D)


def _layernorm(gathered, W_pos_t, W_tt, gamma2d, beta2d):
    n_rows = gathered.shape[0]
    grid = (n_rows // _ROWS_BLK,)
    return pl.pallas_call(
        _ln_body,
        grid=grid,
        in_specs=[
            pl.BlockSpec((_ROWS_BLK, _D), lambda i: (i, 0)),
            pl.BlockSpec((_CHUNK, _D), lambda i: (0, 0)),
            pl.BlockSpec((2, _D), lambda i: (0, 0)),
            pl.BlockSpec((1, _D), lambda i: (0, 0)),
            pl.BlockSpec((1, _D), lambda i: (0, 0)),
        ],
        out_specs=pl.BlockSpec((_ROWS_BLK, _D), lambda i: (i, 0)),
        out_shape=jax.ShapeDtypeStruct((n_rows, _D), jnp.float32),
    )(gathered, W_pos_t, W_tt, gamma2d, beta2d)


def kernel(input_ids, W_word, W_pos, W_tt, gamma, beta):
    B, T = input_ids.shape
    ids_flat = input_ids.reshape(-1).astype(jnp.int32)
    gathered = _make_sc_gather(B * T)(ids_flat, W_word)
    out = _layernorm(
        gathered,
        W_pos[:T],
        W_tt,
        gamma.reshape(1, _D),
        beta.reshape(1, _D),
    )
    return out.reshape(B, T, _D)
